# baseline (speedup 1.0000x reference)
import jax
import jax.numpy as jnp
from jax import lax
from jax.experimental import pallas as pl

N = 2048
BJ = 256
NEG_SLOPE = 0.2


def _gat_block(xc_ref, xr_ref, w_ref, as_ref, ad_ref, b_ref, out_ref):
    w = w_ref[0, :]
    s = jnp.sum(w * as_ref[0, :])
    t = jnp.sum(w * ad_ref[0, :])
    wbar = jnp.mean(w)
    bbar = jnp.mean(b_ref[0, :])

    xc = xc_ref[:, :]
    u = s * xc
    umax = jnp.max(u)
    q = jnp.exp(u - umax)
    r = jnp.exp(NEG_SLOPE * (u - umax))
    v4 = jnp.concatenate([q, q * xc, r, r * xc], axis=1)

    c = t * xr_ref[:, :]
    mask = jnp.where(u + c > 0, 1.0, 0.0)
    sums = lax.dot_general(v4, mask, (((0,), (0,)), ((), ())),
                           preferred_element_type=jnp.float32)
    a = sums[0:1, :]
    ax = sums[1:2, :]
    ar = sums[2:3, :]
    arx = sums[3:4, :]
    rtot = jnp.sum(r)
    rxtot = jnp.sum(r * xc)
    bsum = rtot - ar
    bxsum = rxtot - arx

    g = umax + c
    m = jnp.where(g > 0, g, NEG_SLOPE * g)
    f1 = jnp.exp(g - m)
    f2 = jnp.exp(NEG_SLOPE * g - m)
    denom = f1 * a + f2 * bsum + 1e-16
    numer = f1 * ax + f2 * bxsum
    out_ref[:, :] = wbar * numer / denom + bbar


def kernel(x, W, att_src, att_dst, bias):
    a, b, n, d = x.shape
    xf = x.reshape(n, 1)
    xr = x.reshape(1, n)
    w2 = W.reshape(1, -1)
    as2 = att_src.reshape(1, -1)
    ad2 = att_dst.reshape(1, -1)
    b2 = bias.reshape(1, -1)

    out = pl.pallas_call(
        _gat_block,
        grid=(n // BJ,),
        in_specs=[
            pl.BlockSpec((n, 1), lambda j: (0, 0)),
            pl.BlockSpec((1, BJ), lambda j: (0, j)),
            pl.BlockSpec(w2.shape, lambda j: (0, 0)),
            pl.BlockSpec(as2.shape, lambda j: (0, 0)),
            pl.BlockSpec(ad2.shape, lambda j: (0, 0)),
            pl.BlockSpec(b2.shape, lambda j: (0, 0)),
        ],
        out_specs=pl.BlockSpec((1, BJ), lambda j: (0, j)),
        out_shape=jax.ShapeDtypeStruct((1, n), jnp.float32),
    )(xf, xr, w2, as2, ad2, b2)

    return out.reshape(n, a, b, d).transpose(1, 2, 0, 3)


# speedup vs baseline: 3.1163x; 3.1163x over previous
import jax
import jax.numpy as jnp
from jax import lax
from jax.experimental import pallas as pl

N = 2048
NEG_SLOPE = 0.2


def _gat_kernel(xc_ref, xr_ref, w_ref, as_ref, ad_ref, b_ref, out_ref):
    w = w_ref[0, :]
    s = jnp.sum(w * as_ref[0, :])
    t = jnp.sum(w * ad_ref[0, :])
    wbar = jnp.mean(w)
    bbar = jnp.mean(b_ref[0, :])

    xrow = xr_ref[:, :]
    u = s * xrow
    umax = jnp.max(u)
    q = jnp.exp(u - umax)
    r = jnp.exp(NEG_SLOPE * (u - umax))
    v4 = jnp.concatenate([q, q * xrow, r, r * xrow], axis=0)
    rtot = jnp.sum(v4[2:3, :])
    rxtot = jnp.sum(v4[3:4, :])

    ucol = s * xc_ref[:, :]
    thr = (-t) * xrow
    mask = jnp.where(ucol > thr, 1.0, 0.0)
    sums = lax.dot_general(v4, mask, (((1,), (0,)), ((), ())),
                           preferred_element_type=jnp.float32)
    a = sums[0:1, :]
    ax = sums[1:2, :]
    bsum = rtot - sums[2:3, :]
    bxsum = rxtot - sums[3:4, :]

    g = umax + t * xrow
    m = jnp.where(g > 0, g, NEG_SLOPE * g)
    f1 = jnp.exp(g - m)
    f2 = jnp.exp(NEG_SLOPE * g - m)
    denom = f1 * a + f2 * bsum + 1e-16
    numer = f1 * ax + f2 * bxsum
    out_ref[:, :] = wbar * numer / denom + bbar


def kernel(x, W, att_src, att_dst, bias):
    a, b, n, d = x.shape
    xf = x.reshape(n, 1)
    xr = x.reshape(1, n)
    w2 = W.reshape(1, -1)
    as2 = att_src.reshape(1, -1)
    ad2 = att_dst.reshape(1, -1)
    b2 = bias.reshape(1, -1)

    out = pl.pallas_call(
        _gat_kernel,
        out_shape=jax.ShapeDtypeStruct((1, n), jnp.float32),
    )(xf, xr, w2, as2, ad2, b2)

    return out.reshape(n, a, b, d).transpose(1, 2, 0, 3)
